# Initial kernel scaffold; baseline (speedup 1.0000x reference)
import jax
import jax.numpy as jnp
from jax.experimental import pallas as pl


def kernel(def_sens, embed_weight):
    raise NotImplementedError("write your pallas kernel here")


# speedup vs baseline: 4.8109x; 4.8109x over previous
import functools

import jax
import jax.numpy as jnp
from jax import lax
from jax.experimental import pallas as pl
from jax.experimental.pallas import tpu as pltpu
from jax.experimental.pallas import tpu_sc as plsc

_NUM_WORKERS = 32
_CHUNK = 1024


@functools.partial(jax.jit, static_argnums=(2, 3))
def _gather_flat(idx_flat, table, n, d):
    per_w = n // _NUM_WORKERS
    n_chunks = per_w // _CHUNK
    mesh = plsc.VectorSubcoreMesh(core_axis_name="c", subcore_axis_name="s")

    @functools.partial(
        pl.kernel,
        out_type=jax.ShapeDtypeStruct((n, d), jnp.float32),
        mesh=mesh,
        scratch_types=[
            pltpu.VMEM((_CHUNK,), jnp.int32),
            pltpu.VMEM((_CHUNK, d), jnp.float32),
            pltpu.SemaphoreType.DMA,
        ],
        compiler_params=pltpu.CompilerParams(use_tc_tiling_on_sc=False),
    )
    def gather_kernel(idx_hbm, table_hbm, out_hbm, idx_v, rows_v, sem):
        wid = lax.axis_index("s") * 2 + lax.axis_index("c")
        base = wid * per_w

        def body(i, carry):
            off = base + i * _CHUNK
            pltpu.sync_copy(idx_hbm.at[pl.ds(off, _CHUNK)], idx_v)
            pltpu.async_copy(table_hbm.at[idx_v], rows_v, sem).wait()
            pltpu.sync_copy(rows_v, out_hbm.at[pl.ds(off, _CHUNK)])
            return carry

        lax.fori_loop(0, n_chunks, body, 0)

    return gather_kernel(idx_flat, table)


def kernel(def_sens, embed_weight):
    b, s = def_sens.shape
    v, d = embed_weight.shape
    n = b * s
    out = _gather_flat(def_sens.reshape(n), embed_weight, n, d)
    return out.reshape(b, s, d)


# speedup vs baseline: 5.0259x; 1.0447x over previous
import functools

import jax
import jax.numpy as jnp
from jax import lax
from jax.experimental import pallas as pl
from jax.experimental.pallas import tpu as pltpu
from jax.experimental.pallas import tpu_sc as plsc

_NUM_WORKERS = 32
_CHUNK = 512
_NBUF = 4


@functools.partial(jax.jit, static_argnums=(2, 3))
def _gather_flat(idx_flat, table, n, d):
    per_w = n // _NUM_WORKERS
    group = _NBUF * _CHUNK
    n_groups = per_w // group
    mesh = plsc.VectorSubcoreMesh(core_axis_name="c", subcore_axis_name="s")

    scratch = (
        [pltpu.VMEM((_CHUNK,), jnp.int32) for _ in range(_NBUF)]
        + [pltpu.VMEM((_CHUNK, d), jnp.float32) for _ in range(_NBUF)]
        + [pltpu.SemaphoreType.DMA for _ in range(2 * _NBUF)]
    )

    @functools.partial(
        pl.kernel,
        out_type=jax.ShapeDtypeStruct((n, d), jnp.float32),
        mesh=mesh,
        scratch_types=scratch,
        compiler_params=pltpu.CompilerParams(use_tc_tiling_on_sc=False),
    )
    def gather_kernel(idx_hbm, table_hbm, out_hbm, *bufs):
        idx_v = bufs[:_NBUF]
        rows_v = bufs[_NBUF : 2 * _NBUF]
        gsem = bufs[2 * _NBUF : 3 * _NBUF]
        wsem = bufs[3 * _NBUF :]

        wid = lax.axis_index("s") * 2 + lax.axis_index("c")
        base = wid * per_w

        for b in range(_NBUF):
            pltpu.sync_copy(idx_hbm.at[pl.ds(base + b * _CHUNK, _CHUNK)], idx_v[b])
            pltpu.async_copy(table_hbm.at[idx_v[b]], rows_v[b], gsem[b])

        def outer(g, carry):
            cur = base + g * group
            nxt = cur + group
            for b in range(_NBUF):
                out_slc = out_hbm.at[pl.ds(cur + b * _CHUNK, _CHUNK)]
                pltpu.make_async_copy(table_hbm.at[idx_v[b]], rows_v[b], gsem[b]).wait()
                pltpu.async_copy(rows_v[b], out_slc, wsem[b])
            for b in range(_NBUF):
                out_slc = out_hbm.at[pl.ds(cur + b * _CHUNK, _CHUNK)]
                pltpu.sync_copy(idx_hbm.at[pl.ds(nxt + b * _CHUNK, _CHUNK)], idx_v[b])
                pltpu.make_async_copy(rows_v[b], out_slc, wsem[b]).wait()
                pltpu.async_copy(table_hbm.at[idx_v[b]], rows_v[b], gsem[b])
            return carry

        lax.fori_loop(0, n_groups - 1, outer, 0)

        cur = base + (n_groups - 1) * group
        for b in range(_NBUF):
            out_slc = out_hbm.at[pl.ds(cur + b * _CHUNK, _CHUNK)]
            pltpu.make_async_copy(table_hbm.at[idx_v[b]], rows_v[b], gsem[b]).wait()
            pltpu.async_copy(rows_v[b], out_slc, wsem[b])
        for b in range(_NBUF):
            out_slc = out_hbm.at[pl.ds(cur + b * _CHUNK, _CHUNK)]
            pltpu.make_async_copy(rows_v[b], out_slc, wsem[b]).wait()

    return gather_kernel(idx_flat, table)


def kernel(def_sens, embed_weight):
    b, s = def_sens.shape
    v, d = embed_weight.shape
    n = b * s
    out = _gather_flat(def_sens.reshape(n), embed_weight, n, d)
    return out.reshape(b, s, d)
